# Initial kernel scaffold; baseline (speedup 1.0000x reference)
import jax
import jax.numpy as jnp
from jax.experimental import pallas as pl


def kernel(object_ids, shape_table, texture_table):
    raise NotImplementedError("write your pallas kernel here")


# speedup vs baseline: 1.5249x; 1.5249x over previous
import functools

import jax
import jax.numpy as jnp
from jax import lax
from jax.experimental import pallas as pl
from jax.experimental.pallas import tpu as pltpu
from jax.experimental.pallas import tpu_sc as plsc

D = 128
B = 16384
NC = 2
NS = 16
NW = NC * NS
BPW = B // NW
CHUNK = 128
NCHUNK = BPW // CHUNK
ROUNDS = 2
RCHUNK = NCHUNK // ROUNDS
RROWS = BPW // ROUNDS

_mesh = plsc.VectorSubcoreMesh(core_axis_name="c", subcore_axis_name="s",
                               num_cores=NC, num_subcores=NS)


@functools.partial(
    pl.kernel,
    out_type=(jax.ShapeDtypeStruct((B, D), jnp.float32),
              jax.ShapeDtypeStruct((B, D), jnp.float32)),
    mesh=_mesh,
    scratch_types=[
        pltpu.VMEM((NCHUNK, CHUNK), jnp.int32),
        pltpu.VMEM((RROWS, D), jnp.float32),
        pltpu.VMEM((RROWS, D), jnp.float32),
        pltpu.SemaphoreType.DMA,
        pltpu.SemaphoreType.DMA,
    ],
)
def _gather2(ids_hbm, shape_hbm, tex_hbm, out_s_hbm, out_t_hbm,
             idx_v, rows_s, rows_t, sem_s, sem_t):
    wid = lax.axis_index("s") * NC + lax.axis_index("c")
    base = wid * BPW
    pltpu.sync_copy(ids_hbm.at[pl.ds(wid * NCHUNK, NCHUNK)], idx_v)
    for r in range(ROUNDS):
        cps = []
        for j in range(RCHUNK):
            cps.append(pltpu.async_copy(
                shape_hbm.at[idx_v.at[r * RCHUNK + j]],
                rows_s.at[pl.ds(j * CHUNK, CHUNK)], sem_s))
            cps.append(pltpu.async_copy(
                tex_hbm.at[idx_v.at[r * RCHUNK + j]],
                rows_t.at[pl.ds(j * CHUNK, CHUNK)], sem_t))
        for cp in cps:
            cp.wait()
        pltpu.sync_copy(rows_s, out_s_hbm.at[pl.ds(base + r * RROWS, RROWS)])
        pltpu.sync_copy(rows_t, out_t_hbm.at[pl.ds(base + r * RROWS, RROWS)])


def kernel(object_ids, shape_table, texture_table):
    ids2d = object_ids.astype(jnp.int32).reshape(NW * NCHUNK, CHUNK)
    return _gather2(ids2d, shape_table, texture_table)
